# Initial kernel scaffold; baseline (speedup 1.0000x reference)
import functools

import jax
import jax.numpy as jnp
from jax import lax
from jax.experimental import pallas as pl
from jax.experimental.pallas import tpu as pltpu
from jax.experimental.pallas import tpu_sc as plsc

N = 10000
E = 320000
D = 128

NPAD = 10240
NSUB = 16
NCORE = 2
ROWS_PER_SUB = NPAD // NSUB
EPAD = 327680
EROWS = EPAD // 128
ROWS_PER_W = EROWS // (NSUB * NCORE)

_MESH = plsc.VectorSubcoreMesh(core_axis_name="c", subcore_axis_name="s")


@functools.partial(
    pl.kernel,
    out_type=jax.ShapeDtypeStruct((NCORE, NPAD, 16), jnp.float32),
    mesh=_MESH,
    scratch_types=[
        pltpu.VMEM((ROWS_PER_W, 128), jnp.int32),
        pltpu.VMEM((128, 16), jnp.float32),
        pltpu.VMEM_SHARED((NPAD, 16), jnp.float32),
    ],
)
def _deg_kernel(dst_hbm, z16_hbm, out_hbm, dst_v, ones_v, acc):
    c = lax.axis_index("c")
    s = lax.axis_index("s")
    w = s * NCORE + c

    @pl.loop(0, 128)
    def _(i):
        ones_v[i, :] = jnp.ones((16,), jnp.float32)

    pltpu.sync_copy(z16_hbm.at[pl.ds(s * ROWS_PER_SUB, ROWS_PER_SUB)],
                    acc.at[pl.ds(s * ROWS_PER_SUB, ROWS_PER_SUB)])
    pltpu.sync_copy(dst_hbm.at[pl.ds(w * ROWS_PER_W, ROWS_PER_W)], dst_v)
    plsc.subcore_barrier()

    @pl.loop(0, ROWS_PER_W)
    def _(j):
        pltpu.sync_copy(ones_v, acc.at[dst_v.at[j]], add=True)

    plsc.subcore_barrier()
    pltpu.sync_copy(acc.at[pl.ds(s * ROWS_PER_SUB, ROWS_PER_SUB)],
                    out_hbm.at[c, pl.ds(s * ROWS_PER_SUB, ROWS_PER_SUB)])


def _make_agg(wd):
    @functools.partial(
        pl.kernel,
        out_type=jax.ShapeDtypeStruct((NCORE, NPAD, wd), jnp.float32),
        mesh=_MESH,
        scratch_types=[
            pltpu.VMEM((ROWS_PER_W, 128), jnp.int32),
            pltpu.VMEM((ROWS_PER_W, 128), jnp.int32),
            pltpu.VMEM((128, wd), jnp.float32),
            pltpu.VMEM_SHARED((NPAD, wd), jnp.float32),
        ],
    )
    def _agg(tbl_hbm, src_hbm, dst_hbm, z_hbm, out_hbm,
             src_v, dst_v, rows_v, acc):
        c = lax.axis_index("c")
        s = lax.axis_index("s")
        w = s * NCORE + c
        pltpu.sync_copy(z_hbm.at[pl.ds(s * ROWS_PER_SUB, ROWS_PER_SUB)],
                        acc.at[pl.ds(s * ROWS_PER_SUB, ROWS_PER_SUB)])
        pltpu.sync_copy(src_hbm.at[pl.ds(w * ROWS_PER_W, ROWS_PER_W)], src_v)
        pltpu.sync_copy(dst_hbm.at[pl.ds(w * ROWS_PER_W, ROWS_PER_W)], dst_v)
        plsc.subcore_barrier()

        @pl.loop(0, ROWS_PER_W)
        def _(j):
            pltpu.sync_copy(tbl_hbm.at[src_v.at[j]], rows_v)
            pltpu.sync_copy(rows_v, acc.at[dst_v.at[j]], add=True)

        plsc.subcore_barrier()
        pltpu.sync_copy(acc.at[pl.ds(s * ROWS_PER_SUB, ROWS_PER_SUB)],
                        out_hbm.at[c, pl.ds(s * ROWS_PER_SUB, ROWS_PER_SUB)])

    return _agg


_agg128 = _make_agg(128)
_agg16 = _make_agg(16)


_B = 1024
_GRID = NPAD // _B


def _tc1_body(d0, d1, x_ref, w1, g1_ref, dv_ref):
    deg = d0[:, 0:1] + d1[:, 0:1] + 1.0
    dinv = lax.rsqrt(deg)
    h = jnp.dot(x_ref[...], w1[...], preferred_element_type=jnp.float32)
    g1_ref[...] = h * dinv
    dv_ref[...] = jnp.broadcast_to(dinv, (_B, 16))


def _tc1(d0, d1, xpad, W1):
    return pl.pallas_call(
        _tc1_body,
        grid=(_GRID,),
        in_specs=[
            pl.BlockSpec((_B, 16), lambda i: (i, 0)),
            pl.BlockSpec((_B, 16), lambda i: (i, 0)),
            pl.BlockSpec((_B, D), lambda i: (i, 0)),
            pl.BlockSpec((D, D), lambda i: (0, 0)),
        ],
        out_specs=[
            pl.BlockSpec((_B, D), lambda i: (i, 0)),
            pl.BlockSpec((_B, 16), lambda i: (i, 0)),
        ],
        out_shape=[
            jax.ShapeDtypeStruct((NPAD, D), jnp.float32),
            jax.ShapeDtypeStruct((NPAD, 16), jnp.float32),
        ],
    )(d0, d1, xpad, W1)


def _tc2_body(a0, a1, g1, dv, b1, w2, g2_ref):
    dinv = dv[:, 0:1]
    h = dinv * (a0[...] + a1[...] + g1[...]) + b1[...]
    h = jnp.maximum(h, 0.0)
    g2_ref[...] = dinv * jnp.dot(h, w2[...],
                                 preferred_element_type=jnp.float32)


def _tc2(a0, a1, g1, dv, b1, w2p):
    return pl.pallas_call(
        _tc2_body,
        grid=(_GRID,),
        in_specs=[
            pl.BlockSpec((_B, D), lambda i: (i, 0)),
            pl.BlockSpec((_B, D), lambda i: (i, 0)),
            pl.BlockSpec((_B, D), lambda i: (i, 0)),
            pl.BlockSpec((_B, 16), lambda i: (i, 0)),
            pl.BlockSpec((1, D), lambda i: (0, 0)),
            pl.BlockSpec((D, 16), lambda i: (0, 0)),
        ],
        out_specs=pl.BlockSpec((_B, 16), lambda i: (i, 0)),
        out_shape=jax.ShapeDtypeStruct((NPAD, 16), jnp.float32),
    )(a0, a1, g1, dv, b1, w2p)


def _tc3_body(c0, c1, g2, dv, b2, o_ref):
    o_ref[...] = dv[:, 0:1] * (c0[...] + c1[...] + g2[...]) + b2[...]


def _tc3(c0, c1, g2, dv, b2p):
    return pl.pallas_call(
        _tc3_body,
        grid=(_GRID,),
        in_specs=[
            pl.BlockSpec((_B, 16), lambda i: (i, 0)),
            pl.BlockSpec((_B, 16), lambda i: (i, 0)),
            pl.BlockSpec((_B, 16), lambda i: (i, 0)),
            pl.BlockSpec((_B, 16), lambda i: (i, 0)),
            pl.BlockSpec((1, 16), lambda i: (0, 0)),
        ],
        out_specs=pl.BlockSpec((_B, 16), lambda i: (i, 0)),
        out_shape=jax.ShapeDtypeStruct((NPAD, 16), jnp.float32),
    )(c0, c1, g2, dv, b2p)


def kernel(x, edge_index, W1, b1, W2, b2):
    f32 = jnp.float32
    src = edge_index[0]
    dst = edge_index[1]
    p = EPAD - E
    pad_ids = jnp.arange(p, dtype=jnp.int32)
    srcp = jnp.concatenate([src, (pad_ids * 997) % N]).reshape(EROWS, 128)
    dstp = jnp.concatenate([dst, N + pad_ids % (NPAD - N)]).reshape(EROWS, 128)

    xpad = jnp.zeros((NPAD, D), f32).at[:N].set(x)
    w2p = jnp.zeros((D, 16), f32).at[:, :2].set(W2)
    b2p = jnp.zeros((1, 16), f32).at[0, :2].set(b2)
    z16 = jnp.zeros((NPAD, 16), f32)
    z128 = jnp.zeros((NPAD, D), f32)

    degp = _deg_kernel(dstp, z16)
    g1, dv = _tc1(degp[0], degp[1], xpad, W1)
    acc1 = _agg128(g1, srcp, dstp, z128)
    g2 = _tc2(acc1[0], acc1[1], g1, dv, b1.reshape(1, D), w2p)
    acc2 = _agg16(g2, srcp, dstp, z16)
    o16 = _tc3(acc2[0], acc2[1], g2, dv, b2p)
    return o16[:N, :2]


# speedup vs baseline: 27.4947x; 27.4947x over previous
import functools

import jax
import jax.numpy as jnp
from jax import lax
from jax.experimental import pallas as pl
from jax.experimental.pallas import tpu as pltpu
from jax.experimental.pallas import tpu_sc as plsc

N = 10000
E = 320000
D = 128

NPAD = 10240
NSUB = 16
NCORE = 2
ROWS_PER_SUB = NPAD // NSUB
EPAD = 327680
EROWS = EPAD // 128
ROWS_PER_W = EROWS // (NSUB * NCORE)

_MESH = plsc.VectorSubcoreMesh(core_axis_name="c", subcore_axis_name="s")


@functools.partial(
    pl.kernel,
    out_type=jax.ShapeDtypeStruct((NCORE, NPAD, 16), jnp.float32),
    mesh=_MESH,
    scratch_types=[
        pltpu.VMEM((ROWS_PER_W, 128), jnp.int32),
        pltpu.VMEM((128, 16), jnp.float32),
        pltpu.VMEM_SHARED((NPAD, 16), jnp.float32),
    ],
)
def _deg_kernel(dst_hbm, z16_hbm, out_hbm, dst_v, ones_v, acc):
    c = lax.axis_index("c")
    s = lax.axis_index("s")
    w = s * NCORE + c

    @pl.loop(0, 128)
    def _(i):
        ones_v[i, :] = jnp.ones((16,), jnp.float32)

    pltpu.sync_copy(z16_hbm.at[pl.ds(s * ROWS_PER_SUB, ROWS_PER_SUB)],
                    acc.at[pl.ds(s * ROWS_PER_SUB, ROWS_PER_SUB)])
    pltpu.sync_copy(dst_hbm.at[pl.ds(w * ROWS_PER_W, ROWS_PER_W)], dst_v)
    plsc.subcore_barrier()

    @pl.loop(0, ROWS_PER_W)
    def _(j):
        pltpu.sync_copy(ones_v, acc.at[dst_v.at[j]], add=True)

    plsc.subcore_barrier()
    pltpu.sync_copy(acc.at[pl.ds(s * ROWS_PER_SUB, ROWS_PER_SUB)],
                    out_hbm.at[c, pl.ds(s * ROWS_PER_SUB, ROWS_PER_SUB)])


@functools.partial(
    pl.kernel,
    out_type=jax.ShapeDtypeStruct((NCORE, NPAD, 128), jnp.float32),
    mesh=_MESH,
    scratch_types=[
        pltpu.VMEM((ROWS_PER_W, 128), jnp.int32),
        pltpu.VMEM((ROWS_PER_W, 128), jnp.int32),
        pltpu.VMEM((128, 128), jnp.float32),
        pltpu.VMEM_SHARED((NPAD, 128), jnp.float32),
    ],
)
def _agg128(tbl_hbm, src_hbm, dst_hbm, z_hbm, out_hbm,
            src_v, dst_v, rows_v, acc):
    c = lax.axis_index("c")
    s = lax.axis_index("s")
    w = s * NCORE + c
    pltpu.sync_copy(z_hbm.at[pl.ds(s * ROWS_PER_SUB, ROWS_PER_SUB)],
                    acc.at[pl.ds(s * ROWS_PER_SUB, ROWS_PER_SUB)])
    pltpu.sync_copy(src_hbm.at[pl.ds(w * ROWS_PER_W, ROWS_PER_W)], src_v)
    pltpu.sync_copy(dst_hbm.at[pl.ds(w * ROWS_PER_W, ROWS_PER_W)], dst_v)
    plsc.subcore_barrier()

    @pl.loop(0, ROWS_PER_W)
    def _(j):
        pltpu.sync_copy(tbl_hbm.at[src_v.at[j]], rows_v)
        pltpu.sync_copy(rows_v, acc.at[dst_v.at[j]], add=True)

    plsc.subcore_barrier()
    pltpu.sync_copy(acc.at[pl.ds(s * ROWS_PER_SUB, ROWS_PER_SUB)],
                    out_hbm.at[c, pl.ds(s * ROWS_PER_SUB, ROWS_PER_SUB)])


@functools.partial(
    pl.kernel,
    out_type=jax.ShapeDtypeStruct((NCORE, NPAD, 16), jnp.float32),
    mesh=_MESH,
    scratch_types=[
        pltpu.VMEM((ROWS_PER_W, 128), jnp.int32),
        pltpu.VMEM((ROWS_PER_W, 128), jnp.int32),
        pltpu.VMEM((128, 16), jnp.float32),
        pltpu.VMEM_SHARED((NPAD, 16), jnp.float32),
        pltpu.VMEM_SHARED((NPAD, 16), jnp.float32),
    ],
)
def _agg16(tbl_hbm, src_hbm, dst_hbm, z_hbm, out_hbm,
           src_v, dst_v, rows_v, tbl_sh, acc):
    c = lax.axis_index("c")
    s = lax.axis_index("s")
    w = s * NCORE + c
    sl = pl.ds(s * ROWS_PER_SUB, ROWS_PER_SUB)
    pltpu.sync_copy(z_hbm.at[sl], acc.at[sl])
    pltpu.sync_copy(tbl_hbm.at[sl], tbl_sh.at[sl])
    pltpu.sync_copy(src_hbm.at[pl.ds(w * ROWS_PER_W, ROWS_PER_W)], src_v)
    pltpu.sync_copy(dst_hbm.at[pl.ds(w * ROWS_PER_W, ROWS_PER_W)], dst_v)
    plsc.subcore_barrier()

    @pl.loop(0, ROWS_PER_W)
    def _(j):
        pltpu.sync_copy(tbl_sh.at[src_v.at[j]], rows_v)
        pltpu.sync_copy(rows_v, acc.at[dst_v.at[j]], add=True)

    plsc.subcore_barrier()
    pltpu.sync_copy(acc.at[sl], out_hbm.at[c, sl])


_B = 1024
_GRID = NPAD // _B


def _tc1_body(d0, d1, x_ref, w1, g1_ref, dv_ref):
    deg = d0[:, 0:1] + d1[:, 0:1] + 1.0
    dinv = lax.rsqrt(deg)
    h = jnp.dot(x_ref[...], w1[...], preferred_element_type=jnp.float32)
    g1_ref[...] = h * dinv
    dv_ref[...] = jnp.broadcast_to(dinv, (_B, 16))


def _tc1(d0, d1, xpad, W1):
    return pl.pallas_call(
        _tc1_body,
        grid=(_GRID,),
        in_specs=[
            pl.BlockSpec((_B, 16), lambda i: (i, 0)),
            pl.BlockSpec((_B, 16), lambda i: (i, 0)),
            pl.BlockSpec((_B, D), lambda i: (i, 0)),
            pl.BlockSpec((D, D), lambda i: (0, 0)),
        ],
        out_specs=[
            pl.BlockSpec((_B, D), lambda i: (i, 0)),
            pl.BlockSpec((_B, 16), lambda i: (i, 0)),
        ],
        out_shape=[
            jax.ShapeDtypeStruct((NPAD, D), jnp.float32),
            jax.ShapeDtypeStruct((NPAD, 16), jnp.float32),
        ],
    )(d0, d1, xpad, W1)


def _tc2_body(a0, a1, g1, dv, b1, w2, g2_ref):
    dinv = dv[:, 0:1]
    h = dinv * (a0[...] + a1[...] + g1[...]) + b1[...]
    h = jnp.maximum(h, 0.0)
    g2_ref[...] = dinv * jnp.dot(h, w2[...],
                                 preferred_element_type=jnp.float32)


def _tc2(a0, a1, g1, dv, b1, w2p):
    return pl.pallas_call(
        _tc2_body,
        grid=(_GRID,),
        in_specs=[
            pl.BlockSpec((_B, D), lambda i: (i, 0)),
            pl.BlockSpec((_B, D), lambda i: (i, 0)),
            pl.BlockSpec((_B, D), lambda i: (i, 0)),
            pl.BlockSpec((_B, 16), lambda i: (i, 0)),
            pl.BlockSpec((1, D), lambda i: (0, 0)),
            pl.BlockSpec((D, 16), lambda i: (0, 0)),
        ],
        out_specs=pl.BlockSpec((_B, 16), lambda i: (i, 0)),
        out_shape=jax.ShapeDtypeStruct((NPAD, 16), jnp.float32),
    )(a0, a1, g1, dv, b1, w2p)


def _tc3_body(c0, c1, g2, dv, b2, o_ref):
    o_ref[...] = dv[:, 0:1] * (c0[...] + c1[...] + g2[...]) + b2[...]


def _tc3(c0, c1, g2, dv, b2p):
    return pl.pallas_call(
        _tc3_body,
        grid=(_GRID,),
        in_specs=[
            pl.BlockSpec((_B, 16), lambda i: (i, 0)),
            pl.BlockSpec((_B, 16), lambda i: (i, 0)),
            pl.BlockSpec((_B, 16), lambda i: (i, 0)),
            pl.BlockSpec((_B, 16), lambda i: (i, 0)),
            pl.BlockSpec((1, 16), lambda i: (0, 0)),
        ],
        out_specs=pl.BlockSpec((_B, 16), lambda i: (i, 0)),
        out_shape=jax.ShapeDtypeStruct((NPAD, 16), jnp.float32),
    )(c0, c1, g2, dv, b2p)


def kernel(x, edge_index, W1, b1, W2, b2):
    f32 = jnp.float32
    src = edge_index[0]
    dst = edge_index[1]
    p = EPAD - E
    pad_ids = jnp.arange(p, dtype=jnp.int32)
    srcp = jnp.concatenate([src, (pad_ids * 997) % N]).reshape(EROWS, 128)
    dstp = jnp.concatenate([dst, N + pad_ids % (NPAD - N)]).reshape(EROWS, 128)

    xpad = jnp.zeros((NPAD, D), f32).at[:N].set(x)
    w2p = jnp.zeros((D, 16), f32).at[:, :2].set(W2)
    b2p = jnp.zeros((1, 16), f32).at[0, :2].set(b2)
    z16 = jnp.zeros((NPAD, 16), f32)
    z128 = jnp.zeros((NPAD, D), f32)

    degp = _deg_kernel(dstp, z16)
    g1, dv = _tc1(degp[0], degp[1], xpad, W1)
    acc1 = _agg128(g1, srcp, dstp, z128)
    g2 = _tc2(acc1[0], acc1[1], g1, dv, b1.reshape(1, D), w2p)
    acc2 = _agg16(g2, srcp, dstp, z16)
    o16 = _tc3(acc2[0], acc2[1], g2, dv, b2p)
    return o16[:N, :2]


# speedup vs baseline: 35.5167x; 1.2918x over previous
import functools

import jax
import jax.numpy as jnp
from jax import lax
from jax.experimental import pallas as pl
from jax.experimental.pallas import tpu as pltpu
from jax.experimental.pallas import tpu_sc as plsc

N = 10000
E = 320000
D = 128

NPAD = 10240
NSUB = 16
NCORE = 2
ROWS_PER_SUB = NPAD // NSUB
EPAD = 327680
EROWS = EPAD // 128
ROWS_PER_W = EROWS // (NSUB * NCORE)

_MESH = plsc.VectorSubcoreMesh(core_axis_name="c", subcore_axis_name="s")


def _agg_ring(tbl, src_v, dst_v, bufs, gsems, ssems, acc, nchunks):
    nbuf = len(bufs)
    nstep = nchunks // nbuf
    for b in range(nbuf):
        pltpu.async_copy(tbl.at[src_v.at[b]], bufs[b], gsems[b])

    @pl.loop(0, nstep)
    def _(t):
        for b in range(nbuf):
            j = t * nbuf + b
            pltpu.make_async_copy(tbl.at[src_v.at[j]], bufs[b],
                                  gsems[b]).wait()
            pltpu.async_copy(bufs[b], acc.at[dst_v.at[j]], ssems[b], add=True)

            @pl.when(t < nstep - 1)
            def _():
                pltpu.make_async_copy(bufs[b], acc.at[dst_v.at[j]],
                                      ssems[b]).wait()
                pltpu.async_copy(tbl.at[src_v.at[j + nbuf]], bufs[b], gsems[b])

    for b in range(nbuf):
        pltpu.make_async_copy(bufs[b], acc.at[dst_v.at[nchunks - nbuf + b]],
                              ssems[b]).wait()


@functools.partial(
    pl.kernel,
    out_type=jax.ShapeDtypeStruct((NCORE, NPAD, 16), jnp.float32),
    mesh=_MESH,
    scratch_types=[
        pltpu.VMEM((ROWS_PER_W, 128), jnp.int32),
        pltpu.VMEM((128, 16), jnp.float32),
        pltpu.VMEM_SHARED((NPAD, 16), jnp.float32),
    ] + [pltpu.SemaphoreType.DMA] * 4,
)
def _deg_kernel(dst_hbm, z16_hbm, out_hbm, dst_v, ones_v, acc, *ssems):
    c = lax.axis_index("c")
    s = lax.axis_index("s")
    w = s * NCORE + c
    sl = pl.ds(s * ROWS_PER_SUB, ROWS_PER_SUB)

    @pl.loop(0, 128)
    def _(i):
        ones_v[i, :] = jnp.ones((16,), jnp.float32)

    pltpu.sync_copy(z16_hbm.at[sl], acc.at[sl])
    pltpu.sync_copy(dst_hbm.at[pl.ds(w * ROWS_PER_W, ROWS_PER_W)], dst_v)
    plsc.subcore_barrier()

    nbuf = 4
    for b in range(nbuf):
        pltpu.async_copy(ones_v, acc.at[dst_v.at[b]], ssems[b], add=True)

    @pl.loop(0, ROWS_PER_W // nbuf - 1)
    def _(t):
        for b in range(nbuf):
            j = t * nbuf + b
            pltpu.make_async_copy(ones_v, acc.at[dst_v.at[j]],
                                  ssems[b]).wait()
            pltpu.async_copy(ones_v, acc.at[dst_v.at[j + nbuf]], ssems[b],
                             add=True)

    for b in range(nbuf):
        pltpu.make_async_copy(ones_v, acc.at[dst_v.at[ROWS_PER_W - nbuf + b]],
                              ssems[b]).wait()

    plsc.subcore_barrier()
    pltpu.sync_copy(acc.at[sl], out_hbm.at[c, sl])


_HALF = ROWS_PER_W // 2


@functools.partial(
    pl.kernel,
    out_type=jax.ShapeDtypeStruct((NCORE, NPAD, 128), jnp.float32),
    mesh=_MESH,
    scratch_types=[
        pltpu.VMEM((_HALF, 128), jnp.int32),
        pltpu.VMEM((_HALF, 128), jnp.int32),
        pltpu.VMEM_SHARED((NPAD, 128), jnp.float32),
        pltpu.VMEM((128, 128), jnp.float32),
        pltpu.VMEM((128, 128), jnp.float32),
    ] + [pltpu.SemaphoreType.DMA] * 4,
)
def _agg128(tbl_hbm, src_hbm, dst_hbm, z_hbm, out_hbm,
            src_v, dst_v, acc, buf0, buf1, *sems):
    bufs = (buf0, buf1)
    gsems, ssems = sems[:2], sems[2:]
    c = lax.axis_index("c")
    s = lax.axis_index("s")
    w = s * NCORE + c
    sl = pl.ds(s * ROWS_PER_SUB, ROWS_PER_SUB)
    pltpu.sync_copy(z_hbm.at[sl], acc.at[sl])
    pltpu.sync_copy(src_hbm.at[pl.ds(w * ROWS_PER_W, _HALF)], src_v)
    pltpu.sync_copy(dst_hbm.at[pl.ds(w * ROWS_PER_W, _HALF)], dst_v)
    plsc.subcore_barrier()

    _agg_ring(tbl_hbm, src_v, dst_v, bufs, gsems, ssems, acc, _HALF)
    pltpu.sync_copy(src_hbm.at[pl.ds(w * ROWS_PER_W + _HALF, _HALF)], src_v)
    pltpu.sync_copy(dst_hbm.at[pl.ds(w * ROWS_PER_W + _HALF, _HALF)], dst_v)
    _agg_ring(tbl_hbm, src_v, dst_v, bufs, gsems, ssems, acc, _HALF)

    plsc.subcore_barrier()
    pltpu.sync_copy(acc.at[sl], out_hbm.at[c, sl])


@functools.partial(
    pl.kernel,
    out_type=jax.ShapeDtypeStruct((NCORE, NPAD, 16), jnp.float32),
    mesh=_MESH,
    scratch_types=[
        pltpu.VMEM((ROWS_PER_W, 128), jnp.int32),
        pltpu.VMEM((ROWS_PER_W, 128), jnp.int32),
        pltpu.VMEM_SHARED((NPAD, 16), jnp.float32),
        pltpu.VMEM_SHARED((NPAD, 16), jnp.float32),
    ] + [pltpu.VMEM((128, 16), jnp.float32)] * 4
      + [pltpu.SemaphoreType.DMA] * 8,
)
def _agg16(tbl_hbm, src_hbm, dst_hbm, z_hbm, out_hbm,
           src_v, dst_v, tbl_sh, acc, *rest):
    bufs = rest[:4]
    gsems, ssems = rest[4:8], rest[8:]
    c = lax.axis_index("c")
    s = lax.axis_index("s")
    w = s * NCORE + c
    sl = pl.ds(s * ROWS_PER_SUB, ROWS_PER_SUB)
    pltpu.sync_copy(z_hbm.at[sl], acc.at[sl])
    pltpu.sync_copy(tbl_hbm.at[sl], tbl_sh.at[sl])
    pltpu.sync_copy(src_hbm.at[pl.ds(w * ROWS_PER_W, ROWS_PER_W)], src_v)
    pltpu.sync_copy(dst_hbm.at[pl.ds(w * ROWS_PER_W, ROWS_PER_W)], dst_v)
    plsc.subcore_barrier()

    _agg_ring(tbl_sh, src_v, dst_v, bufs, gsems, ssems, acc, ROWS_PER_W)

    plsc.subcore_barrier()
    pltpu.sync_copy(acc.at[sl], out_hbm.at[c, sl])


_B = 1024
_GRID = NPAD // _B


def _tc1_body(d0, d1, x_ref, w1, g1_ref, dv_ref):
    deg = d0[:, 0:1] + d1[:, 0:1] + 1.0
    dinv = lax.rsqrt(deg)
    h = jnp.dot(x_ref[...], w1[...], preferred_element_type=jnp.float32)
    g1_ref[...] = h * dinv
    dv_ref[...] = jnp.broadcast_to(dinv, (_B, 16))


def _tc1(d0, d1, xpad, W1):
    return pl.pallas_call(
        _tc1_body,
        grid=(_GRID,),
        in_specs=[
            pl.BlockSpec((_B, 16), lambda i: (i, 0)),
            pl.BlockSpec((_B, 16), lambda i: (i, 0)),
            pl.BlockSpec((_B, D), lambda i: (i, 0)),
            pl.BlockSpec((D, D), lambda i: (0, 0)),
        ],
        out_specs=[
            pl.BlockSpec((_B, D), lambda i: (i, 0)),
            pl.BlockSpec((_B, 16), lambda i: (i, 0)),
        ],
        out_shape=[
            jax.ShapeDtypeStruct((NPAD, D), jnp.float32),
            jax.ShapeDtypeStruct((NPAD, 16), jnp.float32),
        ],
    )(d0, d1, xpad, W1)


def _tc2_body(a0, a1, g1, dv, b1, w2, g2_ref):
    dinv = dv[:, 0:1]
    h = dinv * (a0[...] + a1[...] + g1[...]) + b1[...]
    h = jnp.maximum(h, 0.0)
    g2_ref[...] = dinv * jnp.dot(h, w2[...],
                                 preferred_element_type=jnp.float32)


def _tc2(a0, a1, g1, dv, b1, w2p):
    return pl.pallas_call(
        _tc2_body,
        grid=(_GRID,),
        in_specs=[
            pl.BlockSpec((_B, D), lambda i: (i, 0)),
            pl.BlockSpec((_B, D), lambda i: (i, 0)),
            pl.BlockSpec((_B, D), lambda i: (i, 0)),
            pl.BlockSpec((_B, 16), lambda i: (i, 0)),
            pl.BlockSpec((1, D), lambda i: (0, 0)),
            pl.BlockSpec((D, 16), lambda i: (0, 0)),
        ],
        out_specs=pl.BlockSpec((_B, 16), lambda i: (i, 0)),
        out_shape=jax.ShapeDtypeStruct((NPAD, 16), jnp.float32),
    )(a0, a1, g1, dv, b1, w2p)


def _tc3_body(c0, c1, g2, dv, b2, o_ref):
    o_ref[...] = dv[:, 0:1] * (c0[...] + c1[...] + g2[...]) + b2[...]


def _tc3(c0, c1, g2, dv, b2p):
    return pl.pallas_call(
        _tc3_body,
        grid=(_GRID,),
        in_specs=[
            pl.BlockSpec((_B, 16), lambda i: (i, 0)),
            pl.BlockSpec((_B, 16), lambda i: (i, 0)),
            pl.BlockSpec((_B, 16), lambda i: (i, 0)),
            pl.BlockSpec((_B, 16), lambda i: (i, 0)),
            pl.BlockSpec((1, 16), lambda i: (0, 0)),
        ],
        out_specs=pl.BlockSpec((_B, 16), lambda i: (i, 0)),
        out_shape=jax.ShapeDtypeStruct((NPAD, 16), jnp.float32),
    )(c0, c1, g2, dv, b2p)


def kernel(x, edge_index, W1, b1, W2, b2):
    f32 = jnp.float32
    src = edge_index[0]
    dst = edge_index[1]
    p = EPAD - E
    pad_ids = jnp.arange(p, dtype=jnp.int32)
    srcp = jnp.concatenate([src, (pad_ids * 997) % N]).reshape(EROWS, 128)
    dstp = jnp.concatenate([dst, N + pad_ids % (NPAD - N)]).reshape(EROWS, 128)

    xpad = jnp.zeros((NPAD, D), f32).at[:N].set(x)
    w2p = jnp.zeros((D, 16), f32).at[:, :2].set(W2)
    b2p = jnp.zeros((1, 16), f32).at[0, :2].set(b2)
    z16 = jnp.zeros((NPAD, 16), f32)
    z128 = jnp.zeros((NPAD, D), f32)

    degp = _deg_kernel(dstp, z16)
    g1, dv = _tc1(degp[0], degp[1], xpad, W1)
    acc1 = _agg128(g1, srcp, dstp, z128)
    g2 = _tc2(acc1[0], acc1[1], g1, dv, b1.reshape(1, D), w2p)
    acc2 = _agg16(g2, srcp, dstp, z16)
    o16 = _tc3(acc2[0], acc2[1], g2, dv, b2p)
    return o16[:N, :2]


# speedup vs baseline: 36.4194x; 1.0254x over previous
import functools

import jax
import jax.numpy as jnp
from jax import lax
from jax.experimental import pallas as pl
from jax.experimental.pallas import tpu as pltpu
from jax.experimental.pallas import tpu_sc as plsc

N = 10000
E = 320000
D = 128

NPAD = 10240
NSUB = 16
NCORE = 2
ROWS_PER_SUB = NPAD // NSUB
EPAD = 327680
EROWS = EPAD // 128
ROWS_PER_W = EROWS // (NSUB * NCORE)

_MESH = plsc.VectorSubcoreMesh(core_axis_name="c", subcore_axis_name="s")


def _agg_ring(tbl, src_v, dst_v, bufs, gsems, ssems, acc, nchunks):
    nbuf = len(bufs)
    nstep = nchunks // nbuf
    for b in range(nbuf):
        pltpu.async_copy(tbl.at[src_v.at[b]], bufs[b], gsems[b])

    @pl.loop(0, nstep)
    def _(t):
        for b in range(nbuf):
            j = t * nbuf + b
            pltpu.make_async_copy(tbl.at[src_v.at[j]], bufs[b],
                                  gsems[b]).wait()
            pltpu.async_copy(bufs[b], acc.at[dst_v.at[j]], ssems[b], add=True)

            @pl.when(t < nstep - 1)
            def _():
                pltpu.make_async_copy(bufs[b], acc.at[dst_v.at[j]],
                                      ssems[b]).wait()
                pltpu.async_copy(tbl.at[src_v.at[j + nbuf]], bufs[b], gsems[b])

    for b in range(nbuf):
        pltpu.make_async_copy(bufs[b], acc.at[dst_v.at[nchunks - nbuf + b]],
                              ssems[b]).wait()


@functools.partial(
    pl.kernel,
    out_type=jax.ShapeDtypeStruct((NCORE, NPAD, 16), jnp.float32),
    mesh=_MESH,
    scratch_types=[
        pltpu.VMEM((ROWS_PER_W, 128), jnp.int32),
        pltpu.VMEM((128, 16), jnp.float32),
        pltpu.VMEM_SHARED((NPAD, 16), jnp.float32),
    ] + [pltpu.SemaphoreType.DMA] * 4,
)
def _deg_kernel(dst_hbm, z16_hbm, out_hbm, dst_v, ones_v, acc, *ssems):
    c = lax.axis_index("c")
    s = lax.axis_index("s")
    w = s * NCORE + c
    sl = pl.ds(s * ROWS_PER_SUB, ROWS_PER_SUB)

    @pl.loop(0, 128)
    def _(i):
        ones_v[i, :] = jnp.ones((16,), jnp.float32)

    pltpu.sync_copy(z16_hbm.at[sl], acc.at[sl])
    pltpu.sync_copy(dst_hbm.at[pl.ds(w * ROWS_PER_W, ROWS_PER_W)], dst_v)
    plsc.subcore_barrier()

    nbuf = 4
    for b in range(nbuf):
        pltpu.async_copy(ones_v, acc.at[dst_v.at[b]], ssems[b], add=True)

    @pl.loop(0, ROWS_PER_W // nbuf - 1)
    def _(t):
        for b in range(nbuf):
            j = t * nbuf + b
            pltpu.make_async_copy(ones_v, acc.at[dst_v.at[j]],
                                  ssems[b]).wait()
            pltpu.async_copy(ones_v, acc.at[dst_v.at[j + nbuf]], ssems[b],
                             add=True)

    for b in range(nbuf):
        pltpu.make_async_copy(ones_v, acc.at[dst_v.at[ROWS_PER_W - nbuf + b]],
                              ssems[b]).wait()

    plsc.subcore_barrier()
    pltpu.sync_copy(acc.at[sl], out_hbm.at[c, sl])


_C1 = 64
_E1ROWS = EPAD // _C1
_CHUNKS_W = _E1ROWS // (NSUB * NCORE)
_QTR = _CHUNKS_W // 4


@functools.partial(
    pl.kernel,
    out_type=jax.ShapeDtypeStruct((NCORE, NPAD, 128), jnp.float32),
    mesh=_MESH,
    scratch_types=[
        pltpu.VMEM((_QTR, _C1), jnp.int32),
        pltpu.VMEM((_QTR, _C1), jnp.int32),
        pltpu.VMEM_SHARED((NPAD, 128), jnp.float32),
    ] + [pltpu.VMEM((_C1, 128), jnp.float32)] * 4
      + [pltpu.SemaphoreType.DMA] * 8,
)
def _agg128(tbl_hbm, src_hbm, dst_hbm, z_hbm, out_hbm,
            src_v, dst_v, acc, *rest):
    bufs = rest[:4]
    gsems, ssems = rest[4:8], rest[8:]
    c = lax.axis_index("c")
    s = lax.axis_index("s")
    w = s * NCORE + c
    sl = pl.ds(s * ROWS_PER_SUB, ROWS_PER_SUB)
    pltpu.sync_copy(z_hbm.at[sl], acc.at[sl])
    pltpu.sync_copy(src_hbm.at[pl.ds(w * _CHUNKS_W, _QTR)], src_v)
    pltpu.sync_copy(dst_hbm.at[pl.ds(w * _CHUNKS_W, _QTR)], dst_v)
    plsc.subcore_barrier()

    for q in range(4):
        _agg_ring(tbl_hbm, src_v, dst_v, bufs, gsems, ssems, acc, _QTR)
        if q < 3:
            base = w * _CHUNKS_W + (q + 1) * _QTR
            pltpu.sync_copy(src_hbm.at[pl.ds(base, _QTR)], src_v)
            pltpu.sync_copy(dst_hbm.at[pl.ds(base, _QTR)], dst_v)

    plsc.subcore_barrier()
    pltpu.sync_copy(acc.at[sl], out_hbm.at[c, sl])


@functools.partial(
    pl.kernel,
    out_type=jax.ShapeDtypeStruct((NCORE, NPAD, 16), jnp.float32),
    mesh=_MESH,
    scratch_types=[
        pltpu.VMEM((ROWS_PER_W, 128), jnp.int32),
        pltpu.VMEM((ROWS_PER_W, 128), jnp.int32),
        pltpu.VMEM_SHARED((NPAD, 16), jnp.float32),
        pltpu.VMEM_SHARED((NPAD, 16), jnp.float32),
    ] + [pltpu.VMEM((128, 16), jnp.float32)] * 4
      + [pltpu.SemaphoreType.DMA] * 8,
)
def _agg16(tbl_hbm, src_hbm, dst_hbm, z_hbm, out_hbm,
           src_v, dst_v, tbl_sh, acc, *rest):
    bufs = rest[:4]
    gsems, ssems = rest[4:8], rest[8:]
    c = lax.axis_index("c")
    s = lax.axis_index("s")
    w = s * NCORE + c
    sl = pl.ds(s * ROWS_PER_SUB, ROWS_PER_SUB)
    pltpu.sync_copy(z_hbm.at[sl], acc.at[sl])
    pltpu.sync_copy(tbl_hbm.at[sl], tbl_sh.at[sl])
    pltpu.sync_copy(src_hbm.at[pl.ds(w * ROWS_PER_W, ROWS_PER_W)], src_v)
    pltpu.sync_copy(dst_hbm.at[pl.ds(w * ROWS_PER_W, ROWS_PER_W)], dst_v)
    plsc.subcore_barrier()

    _agg_ring(tbl_sh, src_v, dst_v, bufs, gsems, ssems, acc, ROWS_PER_W)

    plsc.subcore_barrier()
    pltpu.sync_copy(acc.at[sl], out_hbm.at[c, sl])


_B = 1024
_GRID = NPAD // _B


def _tc1_body(d0, d1, x_ref, w1, g1_ref, dv_ref):
    deg = d0[:, 0:1] + d1[:, 0:1] + 1.0
    dinv = lax.rsqrt(deg)
    h = jnp.dot(x_ref[...], w1[...], preferred_element_type=jnp.float32)
    g1_ref[...] = h * dinv
    dv_ref[...] = jnp.broadcast_to(dinv, (_B, 16))


def _tc1(d0, d1, xpad, W1):
    return pl.pallas_call(
        _tc1_body,
        grid=(_GRID,),
        in_specs=[
            pl.BlockSpec((_B, 16), lambda i: (i, 0)),
            pl.BlockSpec((_B, 16), lambda i: (i, 0)),
            pl.BlockSpec((_B, D), lambda i: (i, 0)),
            pl.BlockSpec((D, D), lambda i: (0, 0)),
        ],
        out_specs=[
            pl.BlockSpec((_B, D), lambda i: (i, 0)),
            pl.BlockSpec((_B, 16), lambda i: (i, 0)),
        ],
        out_shape=[
            jax.ShapeDtypeStruct((NPAD, D), jnp.float32),
            jax.ShapeDtypeStruct((NPAD, 16), jnp.float32),
        ],
    )(d0, d1, xpad, W1)


def _tc2_body(a0, a1, g1, dv, b1, w2, g2_ref):
    dinv = dv[:, 0:1]
    h = dinv * (a0[...] + a1[...] + g1[...]) + b1[...]
    h = jnp.maximum(h, 0.0)
    g2_ref[...] = dinv * jnp.dot(h, w2[...],
                                 preferred_element_type=jnp.float32)


def _tc2(a0, a1, g1, dv, b1, w2p):
    return pl.pallas_call(
        _tc2_body,
        grid=(_GRID,),
        in_specs=[
            pl.BlockSpec((_B, D), lambda i: (i, 0)),
            pl.BlockSpec((_B, D), lambda i: (i, 0)),
            pl.BlockSpec((_B, D), lambda i: (i, 0)),
            pl.BlockSpec((_B, 16), lambda i: (i, 0)),
            pl.BlockSpec((1, D), lambda i: (0, 0)),
            pl.BlockSpec((D, 16), lambda i: (0, 0)),
        ],
        out_specs=pl.BlockSpec((_B, 16), lambda i: (i, 0)),
        out_shape=jax.ShapeDtypeStruct((NPAD, 16), jnp.float32),
    )(a0, a1, g1, dv, b1, w2p)


def _tc3_body(c0, c1, g2, dv, b2, o_ref):
    o_ref[...] = dv[:, 0:1] * (c0[...] + c1[...] + g2[...]) + b2[...]


def _tc3(c0, c1, g2, dv, b2p):
    return pl.pallas_call(
        _tc3_body,
        grid=(_GRID,),
        in_specs=[
            pl.BlockSpec((_B, 16), lambda i: (i, 0)),
            pl.BlockSpec((_B, 16), lambda i: (i, 0)),
            pl.BlockSpec((_B, 16), lambda i: (i, 0)),
            pl.BlockSpec((_B, 16), lambda i: (i, 0)),
            pl.BlockSpec((1, 16), lambda i: (0, 0)),
        ],
        out_specs=pl.BlockSpec((_B, 16), lambda i: (i, 0)),
        out_shape=jax.ShapeDtypeStruct((NPAD, 16), jnp.float32),
    )(c0, c1, g2, dv, b2p)


def kernel(x, edge_index, W1, b1, W2, b2):
    f32 = jnp.float32
    src = edge_index[0]
    dst = edge_index[1]
    p = EPAD - E
    pad_ids = jnp.arange(p, dtype=jnp.int32)
    srcp = jnp.concatenate([src, (pad_ids * 997) % N]).reshape(EROWS, 128)
    dstp = jnp.concatenate([dst, N + pad_ids % (NPAD - N)]).reshape(EROWS, 128)

    xpad = jnp.zeros((NPAD, D), f32).at[:N].set(x)
    w2p = jnp.zeros((D, 16), f32).at[:, :2].set(W2)
    b2p = jnp.zeros((1, 16), f32).at[0, :2].set(b2)
    z16 = jnp.zeros((NPAD, 16), f32)
    z128 = jnp.zeros((NPAD, D), f32)

    degp = _deg_kernel(dstp, z16)
    g1, dv = _tc1(degp[0], degp[1], xpad, W1)
    acc1 = _agg128(g1, srcp.reshape(_E1ROWS, _C1), dstp.reshape(_E1ROWS, _C1),
                   z128)
    g2 = _tc2(acc1[0], acc1[1], g1, dv, b1.reshape(1, D), w2p)
    acc2 = _agg16(g2, srcp, dstp, z16)
    o16 = _tc3(acc2[0], acc2[1], g2, dv, b2p)
    return o16[:N, :2]


# speedup vs baseline: 41.1609x; 1.1302x over previous
import functools

import jax
import jax.numpy as jnp
from jax import lax
from jax.experimental import pallas as pl
from jax.experimental.pallas import tpu as pltpu
from jax.experimental.pallas import tpu_sc as plsc

N = 10000
E = 320000
D = 128

NPAD = 10240
NSUB = 16
NCORE = 2
ROWS_PER_SUB = NPAD // NSUB
TBL_PER_SUB = N // NSUB
EPAD = 327680
EROWS = EPAD // 128
ROWS_PER_W = EROWS // (NSUB * NCORE)

_MESH = plsc.VectorSubcoreMesh(core_axis_name="c", subcore_axis_name="s")


def _zero_fill(zbuf, acc, row0, nrows, wd):
    @pl.loop(0, 128)
    def _(i):
        for k in range(wd // 16):
            zbuf[i, pl.ds(16 * k, 16)] = jnp.zeros((16,), jnp.float32)

    for m in range(nrows // 128):
        pltpu.sync_copy(zbuf, acc.at[pl.ds(row0 + 128 * m, 128)])


def _agg_ring(tbl, src_v, dst_v, bufs, gsems, ssems, acc, nchunks):
    nbuf = len(bufs)
    nstep = nchunks // nbuf
    for b in range(nbuf):
        pltpu.async_copy(tbl.at[src_v.at[b]], bufs[b], gsems[b])

    @pl.loop(0, nstep)
    def _(t):
        for b in range(nbuf):
            j = t * nbuf + b
            pltpu.make_async_copy(tbl.at[src_v.at[j]], bufs[b],
                                  gsems[b]).wait()
            pltpu.async_copy(bufs[b], acc.at[dst_v.at[j]], ssems[b], add=True)

            @pl.when(t < nstep - 1)
            def _():
                pltpu.make_async_copy(bufs[b], acc.at[dst_v.at[j]],
                                      ssems[b]).wait()
                pltpu.async_copy(tbl.at[src_v.at[j + nbuf]], bufs[b], gsems[b])

    for b in range(nbuf):
        pltpu.make_async_copy(bufs[b], acc.at[dst_v.at[nchunks - nbuf + b]],
                              ssems[b]).wait()


@functools.partial(
    pl.kernel,
    out_type=jax.ShapeDtypeStruct((NCORE, NPAD, 16), jnp.float32),
    mesh=_MESH,
    scratch_types=[
        pltpu.VMEM((ROWS_PER_W, 128), jnp.int32),
        pltpu.VMEM((128, 16), jnp.float32),
        pltpu.VMEM_SHARED((NPAD, 16), jnp.float32),
    ] + [pltpu.SemaphoreType.DMA] * 4,
)
def _deg_kernel(dst_hbm, out_hbm, dst_v, ones_v, acc, *ssems):
    c = lax.axis_index("c")
    s = lax.axis_index("s")
    w = s * NCORE + c
    sl = pl.ds(s * ROWS_PER_SUB, ROWS_PER_SUB)

    _zero_fill(ones_v, acc, s * ROWS_PER_SUB, ROWS_PER_SUB, 16)

    @pl.loop(0, 128)
    def _(i):
        ones_v[i, :] = jnp.ones((16,), jnp.float32)

    pltpu.sync_copy(dst_hbm.at[pl.ds(w * ROWS_PER_W, ROWS_PER_W)], dst_v)
    plsc.subcore_barrier()

    nbuf = 4
    for b in range(nbuf):
        pltpu.async_copy(ones_v, acc.at[dst_v.at[b]], ssems[b], add=True)

    @pl.loop(0, ROWS_PER_W // nbuf - 1)
    def _(t):
        for b in range(nbuf):
            j = t * nbuf + b
            pltpu.make_async_copy(ones_v, acc.at[dst_v.at[j]],
                                  ssems[b]).wait()
            pltpu.async_copy(ones_v, acc.at[dst_v.at[j + nbuf]], ssems[b],
                             add=True)

    for b in range(nbuf):
        pltpu.make_async_copy(ones_v, acc.at[dst_v.at[ROWS_PER_W - nbuf + b]],
                              ssems[b]).wait()

    plsc.subcore_barrier()
    pltpu.sync_copy(acc.at[sl], out_hbm.at[c, sl])


_HALF = ROWS_PER_W // 2


@functools.partial(
    pl.kernel,
    out_type=jax.ShapeDtypeStruct((NCORE, NPAD, 128), jnp.float32),
    mesh=_MESH,
    scratch_types=[
        pltpu.VMEM((_HALF, 128), jnp.int32),
        pltpu.VMEM((_HALF, 128), jnp.int32),
        pltpu.VMEM_SHARED((NPAD, 128), jnp.float32),
        pltpu.VMEM((128, 128), jnp.float32),
        pltpu.VMEM((128, 128), jnp.float32),
    ] + [pltpu.SemaphoreType.DMA] * 4,
)
def _agg128(tbl_hbm, src_hbm, dst_hbm, out_hbm,
            src_v, dst_v, acc, buf0, buf1, *sems):
    bufs = (buf0, buf1)
    gsems, ssems = sems[:2], sems[2:]
    c = lax.axis_index("c")
    s = lax.axis_index("s")
    w = s * NCORE + c
    sl = pl.ds(s * ROWS_PER_SUB, ROWS_PER_SUB)
    _zero_fill(buf0, acc, s * ROWS_PER_SUB, ROWS_PER_SUB, 128)
    pltpu.sync_copy(src_hbm.at[pl.ds(w * ROWS_PER_W, _HALF)], src_v)
    pltpu.sync_copy(dst_hbm.at[pl.ds(w * ROWS_PER_W, _HALF)], dst_v)
    plsc.subcore_barrier()

    _agg_ring(tbl_hbm, src_v, dst_v, bufs, gsems, ssems, acc, _HALF)
    pltpu.sync_copy(src_hbm.at[pl.ds(w * ROWS_PER_W + _HALF, _HALF)], src_v)
    pltpu.sync_copy(dst_hbm.at[pl.ds(w * ROWS_PER_W + _HALF, _HALF)], dst_v)
    _agg_ring(tbl_hbm, src_v, dst_v, bufs, gsems, ssems, acc, _HALF)

    plsc.subcore_barrier()
    pltpu.sync_copy(acc.at[sl], out_hbm.at[c, sl])


@functools.partial(
    pl.kernel,
    out_type=jax.ShapeDtypeStruct((NCORE, NPAD, 16), jnp.float32),
    mesh=_MESH,
    scratch_types=[
        pltpu.VMEM((ROWS_PER_W, 128), jnp.int32),
        pltpu.VMEM((ROWS_PER_W, 128), jnp.int32),
        pltpu.VMEM_SHARED((NPAD, 16), jnp.float32),
        pltpu.VMEM_SHARED((NPAD, 16), jnp.float32),
    ] + [pltpu.VMEM((128, 16), jnp.float32)] * 4
      + [pltpu.SemaphoreType.DMA] * 8,
)
def _agg16(tbl_hbm, src_hbm, dst_hbm, out_hbm,
           src_v, dst_v, tbl_sh, acc, *rest):
    bufs = rest[:4]
    gsems, ssems = rest[4:8], rest[8:]
    c = lax.axis_index("c")
    s = lax.axis_index("s")
    w = s * NCORE + c
    sl = pl.ds(s * ROWS_PER_SUB, ROWS_PER_SUB)
    _zero_fill(bufs[0], acc, s * ROWS_PER_SUB, ROWS_PER_SUB, 16)
    pltpu.sync_copy(tbl_hbm.at[sl], tbl_sh.at[sl])
    pltpu.sync_copy(src_hbm.at[pl.ds(w * ROWS_PER_W, ROWS_PER_W)], src_v)
    pltpu.sync_copy(dst_hbm.at[pl.ds(w * ROWS_PER_W, ROWS_PER_W)], dst_v)
    plsc.subcore_barrier()

    _agg_ring(tbl_sh, src_v, dst_v, bufs, gsems, ssems, acc, ROWS_PER_W)

    plsc.subcore_barrier()
    pltpu.sync_copy(acc.at[sl], out_hbm.at[c, sl])


_B = 1000
_GRID = N // _B


def _tc1_body(dg, x_ref, w1, g1_ref, dv_ref):
    deg = dg[0, :, 0:1] + dg[1, :, 0:1] + 1.0
    dinv = lax.rsqrt(deg)
    h = jnp.dot(x_ref[...].astype(jnp.bfloat16), w1[...].astype(jnp.bfloat16),
                preferred_element_type=jnp.float32)
    g1_ref[...] = h * dinv
    dv_ref[...] = jnp.broadcast_to(dinv, (_B, 16))


def _tc1(degp, x, W1):
    return pl.pallas_call(
        _tc1_body,
        grid=(_GRID,),
        in_specs=[
            pl.BlockSpec((2, _B, 16), lambda i: (0, i, 0)),
            pl.BlockSpec((_B, D), lambda i: (i, 0)),
            pl.BlockSpec((D, D), lambda i: (0, 0)),
        ],
        out_specs=[
            pl.BlockSpec((_B, D), lambda i: (i, 0)),
            pl.BlockSpec((_B, 16), lambda i: (i, 0)),
        ],
        out_shape=[
            jax.ShapeDtypeStruct((N, D), jnp.float32),
            jax.ShapeDtypeStruct((N, 16), jnp.float32),
        ],
    )(degp, x, W1)


def _tc2_body(ac, g1, dv, b1, w2, g2_ref):
    dinv = dv[:, 0:1]
    h = dinv * (ac[0] + ac[1] + g1[...]) + b1[...]
    h = jnp.maximum(h, 0.0)
    g2_ref[...] = dinv * jnp.dot(h.astype(jnp.bfloat16),
                                 w2[...].astype(jnp.bfloat16),
                                 preferred_element_type=jnp.float32)


def _tc2(acc1, g1, dv, b1, w2p):
    return pl.pallas_call(
        _tc2_body,
        grid=(_GRID,),
        in_specs=[
            pl.BlockSpec((2, _B, D), lambda i: (0, i, 0)),
            pl.BlockSpec((_B, D), lambda i: (i, 0)),
            pl.BlockSpec((_B, 16), lambda i: (i, 0)),
            pl.BlockSpec((1, D), lambda i: (0, 0)),
            pl.BlockSpec((D, 16), lambda i: (0, 0)),
        ],
        out_specs=pl.BlockSpec((_B, 16), lambda i: (i, 0)),
        out_shape=jax.ShapeDtypeStruct((NPAD, 16), jnp.float32),
    )(acc1, g1, dv, b1, w2p)


def _tc3_body(ac, g2, dv, b2, o_ref):
    o_ref[...] = dv[:, 0:1] * (ac[0] + ac[1] + g2[...]) + b2[...]


def _tc3(acc2, g2, dv, b2p):
    return pl.pallas_call(
        _tc3_body,
        grid=(_GRID,),
        in_specs=[
            pl.BlockSpec((2, _B, 16), lambda i: (0, i, 0)),
            pl.BlockSpec((_B, 16), lambda i: (i, 0)),
            pl.BlockSpec((_B, 16), lambda i: (i, 0)),
            pl.BlockSpec((1, 16), lambda i: (0, 0)),
        ],
        out_specs=pl.BlockSpec((_B, 16), lambda i: (i, 0)),
        out_shape=jax.ShapeDtypeStruct((N, 16), jnp.float32),
    )(acc2, g2, dv, b2p)


def kernel(x, edge_index, W1, b1, W2, b2):
    f32 = jnp.float32
    src = edge_index[0]
    dst = edge_index[1]
    p = EPAD - E
    pad_ids = jnp.arange(p, dtype=jnp.int32)
    srcp = jnp.concatenate([src, (pad_ids * 997) % N]).reshape(EROWS, 128)
    dstp = jnp.concatenate([dst, N + pad_ids % (NPAD - N)]).reshape(EROWS, 128)

    w2p = jnp.zeros((D, 16), f32).at[:, :2].set(W2)
    b2p = jnp.zeros((1, 16), f32).at[0, :2].set(b2)

    degp = _deg_kernel(dstp)
    g1, dv = _tc1(degp, x, W1)
    acc1 = _agg128(g1, srcp, dstp)
    g2 = _tc2(acc1, g1, dv, b1.reshape(1, D), w2p)
    acc2 = _agg16(g2, srcp, dstp)
    o16 = _tc3(acc2, g2, dv, b2p)
    return o16[:, :2]


# speedup vs baseline: 41.9339x; 1.0188x over previous
import functools

import jax
import jax.numpy as jnp
from jax import lax
from jax.experimental import pallas as pl
from jax.experimental.pallas import tpu as pltpu
from jax.experimental.pallas import tpu_sc as plsc

N = 10000
E = 320000
D = 128

NPAD = 10240
NSUB = 16
NCORE = 2
ROWS_PER_SUB = NPAD // NSUB
TBL_PER_SUB = N // NSUB
EPAD = 327680
EROWS = EPAD // 128
ROWS_PER_W = EROWS // (NSUB * NCORE)

_MESH = plsc.VectorSubcoreMesh(core_axis_name="c", subcore_axis_name="s")


def _zero_fill(zbuf, acc, row0, nrows, wd):
    @pl.loop(0, 128)
    def _(i):
        for k in range(wd // 16):
            zbuf[i, pl.ds(16 * k, 16)] = jnp.zeros((16,), jnp.float32)

    for m in range(nrows // 128):
        pltpu.sync_copy(zbuf, acc.at[pl.ds(row0 + 128 * m, 128)])


def _agg_ring(tbl, src_v, dst_v, bufs, gsems, ssems, acc, nchunks):
    nbuf = len(bufs)
    nstep = nchunks // nbuf
    for b in range(nbuf):
        pltpu.async_copy(tbl.at[src_v.at[b]], bufs[b], gsems[b])

    @pl.loop(0, nstep)
    def _(t):
        for b in range(nbuf):
            j = t * nbuf + b
            pltpu.make_async_copy(tbl.at[src_v.at[j]], bufs[b],
                                  gsems[b]).wait()
            pltpu.async_copy(bufs[b], acc.at[dst_v.at[j]], ssems[b], add=True)

            @pl.when(t < nstep - 1)
            def _():
                pltpu.make_async_copy(bufs[b], acc.at[dst_v.at[j]],
                                      ssems[b]).wait()
                pltpu.async_copy(tbl.at[src_v.at[j + nbuf]], bufs[b], gsems[b])

    for b in range(nbuf):
        pltpu.make_async_copy(bufs[b], acc.at[dst_v.at[nchunks - nbuf + b]],
                              ssems[b]).wait()


@functools.partial(
    pl.kernel,
    out_type=jax.ShapeDtypeStruct((NCORE, NPAD, 16), jnp.float32),
    mesh=_MESH,
    scratch_types=[
        pltpu.VMEM((ROWS_PER_W, 128), jnp.int32),
        pltpu.VMEM((128, 16), jnp.float32),
        pltpu.VMEM_SHARED((NPAD, 16), jnp.float32),
    ] + [pltpu.SemaphoreType.DMA] * 4,
)
def _deg_kernel(dst_hbm, out_hbm, dst_v, ones_v, acc, *ssems):
    c = lax.axis_index("c")
    s = lax.axis_index("s")
    w = s * NCORE + c
    sl = pl.ds(s * ROWS_PER_SUB, ROWS_PER_SUB)

    _zero_fill(ones_v, acc, s * ROWS_PER_SUB, ROWS_PER_SUB, 16)

    @pl.loop(0, 128)
    def _(i):
        ones_v[i, :] = jnp.ones((16,), jnp.float32)

    pltpu.sync_copy(dst_hbm.at[pl.ds(w * ROWS_PER_W, ROWS_PER_W)], dst_v)
    plsc.subcore_barrier()

    nbuf = 4
    for b in range(nbuf):
        pltpu.async_copy(ones_v, acc.at[dst_v.at[b]], ssems[b], add=True)

    @pl.loop(0, ROWS_PER_W // nbuf - 1)
    def _(t):
        for b in range(nbuf):
            j = t * nbuf + b
            pltpu.make_async_copy(ones_v, acc.at[dst_v.at[j]],
                                  ssems[b]).wait()
            pltpu.async_copy(ones_v, acc.at[dst_v.at[j + nbuf]], ssems[b],
                             add=True)

    for b in range(nbuf):
        pltpu.make_async_copy(ones_v, acc.at[dst_v.at[ROWS_PER_W - nbuf + b]],
                              ssems[b]).wait()

    plsc.subcore_barrier()
    pltpu.sync_copy(acc.at[sl], out_hbm.at[c, sl])


_HALF = ROWS_PER_W // 2


@functools.partial(
    pl.kernel,
    out_type=jax.ShapeDtypeStruct((NCORE, NPAD, 128), jnp.float32),
    mesh=_MESH,
    scratch_types=[
        pltpu.VMEM((_HALF, 128), jnp.int32),
        pltpu.VMEM((_HALF, 128), jnp.int32),
        pltpu.VMEM_SHARED((NPAD, 128), jnp.float32),
        pltpu.VMEM((128, 128), jnp.float32),
        pltpu.VMEM((128, 128), jnp.float32),
    ] + [pltpu.SemaphoreType.DMA] * 4,
)
def _agg128(tbl_hbm, src_hbm, dst_hbm, out_hbm,
            src_v, dst_v, acc, buf0, buf1, *sems):
    bufs = (buf0, buf1)
    gsems, ssems = sems[:2], sems[2:]
    c = lax.axis_index("c")
    s = lax.axis_index("s")
    w = s * NCORE + c
    sl = pl.ds(s * ROWS_PER_SUB, ROWS_PER_SUB)
    _zero_fill(buf0, acc, s * ROWS_PER_SUB, ROWS_PER_SUB, 128)
    pltpu.sync_copy(src_hbm.at[pl.ds(w * ROWS_PER_W, _HALF)], src_v)
    pltpu.sync_copy(dst_hbm.at[pl.ds(w * ROWS_PER_W, _HALF)], dst_v)
    plsc.subcore_barrier()

    _agg_ring(tbl_hbm, src_v, dst_v, bufs, gsems, ssems, acc, _HALF)
    pltpu.sync_copy(src_hbm.at[pl.ds(w * ROWS_PER_W + _HALF, _HALF)], src_v)
    pltpu.sync_copy(dst_hbm.at[pl.ds(w * ROWS_PER_W + _HALF, _HALF)], dst_v)
    _agg_ring(tbl_hbm, src_v, dst_v, bufs, gsems, ssems, acc, _HALF)

    plsc.subcore_barrier()
    pltpu.sync_copy(acc.at[sl], out_hbm.at[c, sl])


@functools.partial(
    pl.kernel,
    out_type=jax.ShapeDtypeStruct((NCORE, NPAD, 16), jnp.float32),
    mesh=_MESH,
    scratch_types=[
        pltpu.VMEM((ROWS_PER_W, 128), jnp.int32),
        pltpu.VMEM((ROWS_PER_W, 128), jnp.int32),
        pltpu.VMEM_SHARED((NPAD, 16), jnp.float32),
        pltpu.VMEM_SHARED((NPAD, 16), jnp.float32),
    ] + [pltpu.VMEM((128, 16), jnp.float32)] * 4
      + [pltpu.SemaphoreType.DMA] * 8,
)
def _agg16(tbl_hbm, src_hbm, dst_hbm, out_hbm,
           src_v, dst_v, tbl_sh, acc, *rest):
    bufs = rest[:4]
    gsems, ssems = rest[4:8], rest[8:]
    c = lax.axis_index("c")
    s = lax.axis_index("s")
    w = s * NCORE + c
    sl = pl.ds(s * ROWS_PER_SUB, ROWS_PER_SUB)
    _zero_fill(bufs[0], acc, s * ROWS_PER_SUB, ROWS_PER_SUB, 16)
    pltpu.sync_copy(tbl_hbm.at[sl], tbl_sh.at[sl])
    pltpu.sync_copy(src_hbm.at[pl.ds(w * ROWS_PER_W, ROWS_PER_W)], src_v)
    pltpu.sync_copy(dst_hbm.at[pl.ds(w * ROWS_PER_W, ROWS_PER_W)], dst_v)
    plsc.subcore_barrier()

    _agg_ring(tbl_sh, src_v, dst_v, bufs, gsems, ssems, acc, ROWS_PER_W)

    plsc.subcore_barrier()
    pltpu.sync_copy(acc.at[sl], out_hbm.at[c, sl])


def _tc1_body(dg, x_ref, w1, g1_ref, dv_ref):
    deg = dg[0, :N, 0:1] + dg[1, :N, 0:1] + 1.0
    dinv = lax.rsqrt(deg)
    h = jnp.dot(x_ref[...].astype(jnp.bfloat16), w1[...].astype(jnp.bfloat16),
                preferred_element_type=jnp.float32)
    g1_ref[...] = h * dinv
    dv_ref[...] = jnp.broadcast_to(dinv, (N, 16))


def _tc1(degp, x, W1):
    return pl.pallas_call(
        _tc1_body,
        out_shape=[
            jax.ShapeDtypeStruct((N, D), jnp.float32),
            jax.ShapeDtypeStruct((N, 16), jnp.float32),
        ],
    )(degp, x, W1)


def _tc2_body(ac, g1, dv, b1, w2, g2_ref):
    dinv = dv[:, 0:1]
    h = dinv * (ac[0, :N] + ac[1, :N] + g1[...]) + b1[...]
    h = jnp.maximum(h, 0.0)
    g2_ref[:N, :] = dinv * jnp.dot(h.astype(jnp.bfloat16),
                                   w2[...].astype(jnp.bfloat16),
                                   preferred_element_type=jnp.float32)


def _tc2(acc1, g1, dv, b1, w2p):
    return pl.pallas_call(
        _tc2_body,
        out_shape=jax.ShapeDtypeStruct((NPAD, 16), jnp.float32),
    )(acc1, g1, dv, b1, w2p)


def _tc3_body(ac, g2, dv, b2, o_ref):
    o_ref[...] = dv[:, 0:1] * (ac[0, :N] + ac[1, :N] + g2[:N]) + b2[...]


def _tc3(acc2, g2, dv, b2p):
    return pl.pallas_call(
        _tc3_body,
        out_shape=jax.ShapeDtypeStruct((N, 16), jnp.float32),
    )(acc2, g2, dv, b2p)


def kernel(x, edge_index, W1, b1, W2, b2):
    f32 = jnp.float32
    src = edge_index[0]
    dst = edge_index[1]
    p = EPAD - E
    pad_ids = jnp.arange(p, dtype=jnp.int32)
    srcp = jnp.concatenate([src, (pad_ids * 997) % N]).reshape(EROWS, 128)
    dstp = jnp.concatenate([dst, N + pad_ids % (NPAD - N)]).reshape(EROWS, 128)

    w2p = jnp.zeros((D, 16), f32).at[:, :2].set(W2)
    b2p = jnp.zeros((1, 16), f32).at[0, :2].set(b2)

    degp = _deg_kernel(dstp)
    g1, dv = _tc1(degp, x, W1)
    acc1 = _agg128(g1, srcp, dstp)
    g2 = _tc2(acc1, g1, dv, b1.reshape(1, D), w2p)
    acc2 = _agg16(g2, srcp, dstp)
    o16 = _tc3(acc2, g2, dv, b2p)
    return o16[:, :2]


# speedup vs baseline: 42.4680x; 1.0127x over previous
import functools

import jax
import jax.numpy as jnp
from jax import lax
from jax.experimental import pallas as pl
from jax.experimental.pallas import tpu as pltpu
from jax.experimental.pallas import tpu_sc as plsc

N = 10000
E = 320000
D = 128

NPAD = 10240
NSUB = 16
NCORE = 2
ROWS_PER_SUB = NPAD // NSUB
TBL_PER_SUB = N // NSUB
EPAD = 327680
EROWS = EPAD // 128
ROWS_PER_W = EROWS // (NSUB * NCORE)

_MESH = plsc.VectorSubcoreMesh(core_axis_name="c", subcore_axis_name="s")


def _zero_fill(zbuf, acc, row0, nrows, wd):
    @pl.loop(0, 128)
    def _(i):
        for k in range(wd // 16):
            zbuf[i, pl.ds(16 * k, 16)] = jnp.zeros((16,), jnp.float32)

    for m in range(nrows // 128):
        pltpu.sync_copy(zbuf, acc.at[pl.ds(row0 + 128 * m, 128)])


def _agg_ring(tbl, src_v, dst_v, bufs, gsems, ssems, acc, nchunks):
    nbuf = len(bufs)
    nstep = nchunks // nbuf
    for b in range(nbuf):
        pltpu.async_copy(tbl.at[src_v.at[b]], bufs[b], gsems[b])

    @pl.loop(0, nstep)
    def _(t):
        for b in range(nbuf):
            j = t * nbuf + b
            pltpu.make_async_copy(tbl.at[src_v.at[j]], bufs[b],
                                  gsems[b]).wait()
            pltpu.async_copy(bufs[b], acc.at[dst_v.at[j]], ssems[b], add=True)

            @pl.when(t < nstep - 1)
            def _():
                pltpu.make_async_copy(bufs[b], acc.at[dst_v.at[j]],
                                      ssems[b]).wait()
                pltpu.async_copy(tbl.at[src_v.at[j + nbuf]], bufs[b], gsems[b])

    for b in range(nbuf):
        pltpu.make_async_copy(bufs[b], acc.at[dst_v.at[nchunks - nbuf + b]],
                              ssems[b]).wait()


@functools.partial(
    pl.kernel,
    out_type=jax.ShapeDtypeStruct((NCORE, NPAD, 16), jnp.float32),
    mesh=_MESH,
    scratch_types=[
        pltpu.VMEM((ROWS_PER_W, 128), jnp.int32),
        pltpu.VMEM((128, 16), jnp.float32),
        pltpu.VMEM_SHARED((NPAD, 16), jnp.float32),
    ] + [pltpu.SemaphoreType.DMA] * 4,
)
def _deg_kernel(ei_hbm, out_hbm, dst_v, ones_v, acc, *ssems):
    c = lax.axis_index("c")
    s = lax.axis_index("s")
    w = s * NCORE + c
    sl = pl.ds(s * ROWS_PER_SUB, ROWS_PER_SUB)

    _zero_fill(ones_v, acc, s * ROWS_PER_SUB, ROWS_PER_SUB, 16)

    @pl.loop(0, 128)
    def _(i):
        ones_v[i, :] = jnp.ones((16,), jnp.float32)

    pltpu.sync_copy(ei_hbm.at[1, pl.ds(w * ROWS_PER_W, ROWS_PER_W)], dst_v)
    plsc.subcore_barrier()

    nbuf = 4
    for b in range(nbuf):
        pltpu.async_copy(ones_v, acc.at[dst_v.at[b]], ssems[b], add=True)

    @pl.loop(0, ROWS_PER_W // nbuf - 1)
    def _(t):
        for b in range(nbuf):
            j = t * nbuf + b
            pltpu.make_async_copy(ones_v, acc.at[dst_v.at[j]],
                                  ssems[b]).wait()
            pltpu.async_copy(ones_v, acc.at[dst_v.at[j + nbuf]], ssems[b],
                             add=True)

    for b in range(nbuf):
        pltpu.make_async_copy(ones_v, acc.at[dst_v.at[ROWS_PER_W - nbuf + b]],
                              ssems[b]).wait()

    plsc.subcore_barrier()
    pltpu.sync_copy(acc.at[sl], out_hbm.at[c, sl])


_HALF = ROWS_PER_W // 2


@functools.partial(
    pl.kernel,
    out_type=jax.ShapeDtypeStruct((NCORE, NPAD, 128), jnp.float32),
    mesh=_MESH,
    scratch_types=[
        pltpu.VMEM((_HALF, 128), jnp.int32),
        pltpu.VMEM((_HALF, 128), jnp.int32),
        pltpu.VMEM_SHARED((NPAD, 128), jnp.float32),
        pltpu.VMEM((128, 128), jnp.float32),
        pltpu.VMEM((128, 128), jnp.float32),
    ] + [pltpu.SemaphoreType.DMA] * 4,
)
def _agg128(tbl_hbm, ei_hbm, out_hbm,
            src_v, dst_v, acc, buf0, buf1, *sems):
    bufs = (buf0, buf1)
    gsems, ssems = sems[:2], sems[2:]
    c = lax.axis_index("c")
    s = lax.axis_index("s")
    w = s * NCORE + c
    sl = pl.ds(s * ROWS_PER_SUB, ROWS_PER_SUB)
    _zero_fill(buf0, acc, s * ROWS_PER_SUB, ROWS_PER_SUB, 128)
    pltpu.sync_copy(ei_hbm.at[0, pl.ds(w * ROWS_PER_W, _HALF)], src_v)
    pltpu.sync_copy(ei_hbm.at[1, pl.ds(w * ROWS_PER_W, _HALF)], dst_v)
    plsc.subcore_barrier()

    _agg_ring(tbl_hbm, src_v, dst_v, bufs, gsems, ssems, acc, _HALF)
    pltpu.sync_copy(ei_hbm.at[0, pl.ds(w * ROWS_PER_W + _HALF, _HALF)], src_v)
    pltpu.sync_copy(ei_hbm.at[1, pl.ds(w * ROWS_PER_W + _HALF, _HALF)], dst_v)
    _agg_ring(tbl_hbm, src_v, dst_v, bufs, gsems, ssems, acc, _HALF)

    plsc.subcore_barrier()
    pltpu.sync_copy(acc.at[sl], out_hbm.at[c, sl])


@functools.partial(
    pl.kernel,
    out_type=jax.ShapeDtypeStruct((NCORE, NPAD, 16), jnp.float32),
    mesh=_MESH,
    scratch_types=[
        pltpu.VMEM((ROWS_PER_W, 128), jnp.int32),
        pltpu.VMEM((ROWS_PER_W, 128), jnp.int32),
        pltpu.VMEM_SHARED((NPAD, 16), jnp.float32),
        pltpu.VMEM_SHARED((NPAD, 16), jnp.float32),
    ] + [pltpu.VMEM((128, 16), jnp.float32)] * 4
      + [pltpu.SemaphoreType.DMA] * 8,
)
def _agg16(tbl_hbm, ei_hbm, out_hbm,
           src_v, dst_v, tbl_sh, acc, *rest):
    bufs = rest[:4]
    gsems, ssems = rest[4:8], rest[8:]
    c = lax.axis_index("c")
    s = lax.axis_index("s")
    w = s * NCORE + c
    sl = pl.ds(s * ROWS_PER_SUB, ROWS_PER_SUB)
    _zero_fill(bufs[0], acc, s * ROWS_PER_SUB, ROWS_PER_SUB, 16)
    pltpu.sync_copy(tbl_hbm.at[sl], tbl_sh.at[sl])
    pltpu.sync_copy(ei_hbm.at[0, pl.ds(w * ROWS_PER_W, ROWS_PER_W)], src_v)
    pltpu.sync_copy(ei_hbm.at[1, pl.ds(w * ROWS_PER_W, ROWS_PER_W)], dst_v)
    plsc.subcore_barrier()

    _agg_ring(tbl_sh, src_v, dst_v, bufs, gsems, ssems, acc, ROWS_PER_W)

    plsc.subcore_barrier()
    pltpu.sync_copy(acc.at[sl], out_hbm.at[c, sl])


_B = 1000
_GRID = N // _B


def _tc1_body(dg, x_ref, w1, g1_ref, dv_ref):
    deg = dg[0, :, 0:1] + dg[1, :, 0:1] + 1.0
    dinv = lax.rsqrt(deg)
    h = jnp.dot(x_ref[...].astype(jnp.bfloat16), w1[...].astype(jnp.bfloat16),
                preferred_element_type=jnp.float32)
    g1_ref[...] = h * dinv
    dv_ref[...] = jnp.broadcast_to(dinv, (_B, 16))


def _tc1(degp, x, W1):
    return pl.pallas_call(
        _tc1_body,
        grid=(_GRID,),
        in_specs=[
            pl.BlockSpec((2, _B, 16), lambda i: (0, i, 0)),
            pl.BlockSpec((_B, D), lambda i: (i, 0)),
            pl.BlockSpec((D, D), lambda i: (0, 0)),
        ],
        out_specs=[
            pl.BlockSpec((_B, D), lambda i: (i, 0)),
            pl.BlockSpec((_B, 16), lambda i: (i, 0)),
        ],
        out_shape=[
            jax.ShapeDtypeStruct((N, D), jnp.float32),
            jax.ShapeDtypeStruct((N, 16), jnp.float32),
        ],
    )(degp, x, W1)


def _tc2_body(ac, g1, dv, b1, w2, g2_ref):
    dinv = dv[:, 0:1]
    h = dinv * (ac[0] + ac[1] + g1[...]) + b1[...]
    h = jnp.maximum(h, 0.0)
    g2_ref[...] = dinv * jnp.dot(h.astype(jnp.bfloat16),
                                 w2[...].astype(jnp.bfloat16),
                                 preferred_element_type=jnp.float32)


def _tc2(acc1, g1, dv, b1, w2p):
    return pl.pallas_call(
        _tc2_body,
        grid=(_GRID,),
        in_specs=[
            pl.BlockSpec((2, _B, D), lambda i: (0, i, 0)),
            pl.BlockSpec((_B, D), lambda i: (i, 0)),
            pl.BlockSpec((_B, 16), lambda i: (i, 0)),
            pl.BlockSpec((1, D), lambda i: (0, 0)),
            pl.BlockSpec((D, 16), lambda i: (0, 0)),
        ],
        out_specs=pl.BlockSpec((_B, 16), lambda i: (i, 0)),
        out_shape=jax.ShapeDtypeStruct((NPAD, 16), jnp.float32),
    )(acc1, g1, dv, b1, w2p)


def _tc3_body(ac, g2, dv, b2, o_ref):
    o_ref[...] = dv[:, 0:1] * (ac[0] + ac[1] + g2[...]) + b2[...]


def _tc3(acc2, g2, dv, b2p):
    return pl.pallas_call(
        _tc3_body,
        grid=(_GRID,),
        in_specs=[
            pl.BlockSpec((2, _B, 16), lambda i: (0, i, 0)),
            pl.BlockSpec((_B, 16), lambda i: (i, 0)),
            pl.BlockSpec((_B, 16), lambda i: (i, 0)),
            pl.BlockSpec((1, 16), lambda i: (0, 0)),
        ],
        out_specs=pl.BlockSpec((_B, 16), lambda i: (i, 0)),
        out_shape=jax.ShapeDtypeStruct((N, 16), jnp.float32),
    )(acc2, g2, dv, b2p)


def kernel(x, edge_index, W1, b1, W2, b2):
    f32 = jnp.float32
    p = EPAD - E
    pad_ids = jnp.arange(p, dtype=jnp.int32)
    padblk = jnp.stack([(pad_ids * 997) % N, N + pad_ids % (NPAD - N)])
    ei = jnp.concatenate([edge_index, padblk], axis=1).reshape(2, EROWS, 128)

    w2p = jnp.zeros((D, 16), f32).at[:, :2].set(W2)
    b2p = jnp.zeros((1, 16), f32).at[0, :2].set(b2)

    degp = _deg_kernel(ei)
    g1, dv = _tc1(degp, x, W1)
    acc1 = _agg128(g1, ei)
    g2 = _tc2(acc1, g1, dv, b1.reshape(1, D), w2p)
    acc2 = _agg16(g2, ei)
    o16 = _tc3(acc2, g2, dv, b2p)
    return o16[:, :2]


# speedup vs baseline: 43.6075x; 1.0268x over previous
import functools

import jax
import jax.numpy as jnp
from jax import lax
from jax.experimental import pallas as pl
from jax.experimental.pallas import tpu as pltpu
from jax.experimental.pallas import tpu_sc as plsc

N = 10000
E = 320000
D = 128

NPAD = 10240
NSUB = 16
NCORE = 2
ROWS_PER_SUB = NPAD // NSUB
TBL_PER_SUB = N // NSUB
EPAD = 327680
EROWS = EPAD // 128
ROWS_PER_W = EROWS // (NSUB * NCORE)

_MESH = plsc.VectorSubcoreMesh(core_axis_name="c", subcore_axis_name="s")


def _zero_fill(zbuf, acc, row0, nrows, wd):
    @pl.loop(0, 128)
    def _(i):
        for k in range(wd // 16):
            zbuf[i, pl.ds(16 * k, 16)] = jnp.zeros((16,), jnp.float32)

    for m in range(nrows // 128):
        pltpu.sync_copy(zbuf, acc.at[pl.ds(row0 + 128 * m, 128)])


def _agg_ring(tbl, src_v, dst_v, bufs, gsems, ssems, acc, nchunks):
    nbuf = len(bufs)
    nstep = nchunks // nbuf
    for b in range(nbuf):
        pltpu.async_copy(tbl.at[src_v.at[b]], bufs[b], gsems[b])

    @pl.loop(0, nstep)
    def _(t):
        for b in range(nbuf):
            j = t * nbuf + b
            pltpu.make_async_copy(tbl.at[src_v.at[j]], bufs[b],
                                  gsems[b]).wait()
            pltpu.async_copy(bufs[b], acc.at[dst_v.at[j]], ssems[b], add=True)

            @pl.when(t < nstep - 1)
            def _():
                pltpu.make_async_copy(bufs[b], acc.at[dst_v.at[j]],
                                      ssems[b]).wait()
                pltpu.async_copy(tbl.at[src_v.at[j + nbuf]], bufs[b], gsems[b])

    for b in range(nbuf):
        pltpu.make_async_copy(bufs[b], acc.at[dst_v.at[nchunks - nbuf + b]],
                              ssems[b]).wait()


@functools.partial(
    pl.kernel,
    out_type=jax.ShapeDtypeStruct((NCORE, NPAD, 16), jnp.float32),
    mesh=_MESH,
    scratch_types=[
        pltpu.VMEM((ROWS_PER_W, 128), jnp.int32),
        pltpu.VMEM((128, 16), jnp.float32),
        pltpu.VMEM_SHARED((NPAD, 16), jnp.float32),
    ] + [pltpu.SemaphoreType.DMA] * 4,
)
def _deg_kernel(ei_hbm, out_hbm, dst_v, ones_v, acc, *ssems):
    c = lax.axis_index("c")
    s = lax.axis_index("s")
    w = s * NCORE + c
    sl = pl.ds(s * ROWS_PER_SUB, ROWS_PER_SUB)

    _zero_fill(ones_v, acc, s * ROWS_PER_SUB, ROWS_PER_SUB, 16)

    @pl.loop(0, 128)
    def _(i):
        ones_v[i, :] = jnp.ones((16,), jnp.float32)

    pltpu.sync_copy(ei_hbm.at[1, pl.ds(w * ROWS_PER_W, ROWS_PER_W)], dst_v)
    plsc.subcore_barrier()

    nbuf = 4
    for b in range(nbuf):
        pltpu.async_copy(ones_v, acc.at[dst_v.at[b]], ssems[b], add=True)

    @pl.loop(0, ROWS_PER_W // nbuf - 1)
    def _(t):
        for b in range(nbuf):
            j = t * nbuf + b
            pltpu.make_async_copy(ones_v, acc.at[dst_v.at[j]],
                                  ssems[b]).wait()
            pltpu.async_copy(ones_v, acc.at[dst_v.at[j + nbuf]], ssems[b],
                             add=True)

    for b in range(nbuf):
        pltpu.make_async_copy(ones_v, acc.at[dst_v.at[ROWS_PER_W - nbuf + b]],
                              ssems[b]).wait()

    plsc.subcore_barrier()
    pltpu.sync_copy(acc.at[sl], out_hbm.at[c, sl])


_HALF = ROWS_PER_W // 2


@functools.partial(
    pl.kernel,
    out_type=jax.ShapeDtypeStruct((NCORE, NPAD, 128), jnp.float32),
    mesh=_MESH,
    scratch_types=[
        pltpu.VMEM((_HALF, 128), jnp.int32),
        pltpu.VMEM((_HALF, 128), jnp.int32),
        pltpu.VMEM_SHARED((NPAD, 128), jnp.float32),
        pltpu.VMEM((128, 128), jnp.float32),
        pltpu.VMEM((128, 128), jnp.float32),
    ] + [pltpu.SemaphoreType.DMA] * 4,
)
def _agg128(tbl_hbm, ei_hbm, out_hbm,
            src_v, dst_v, acc, buf0, buf1, *sems):
    bufs = (buf0, buf1)
    gsems, ssems = sems[:2], sems[2:]
    c = lax.axis_index("c")
    s = lax.axis_index("s")
    w = s * NCORE + c
    sl = pl.ds(s * ROWS_PER_SUB, ROWS_PER_SUB)
    _zero_fill(buf0, acc, s * ROWS_PER_SUB, ROWS_PER_SUB, 128)
    pltpu.sync_copy(ei_hbm.at[0, pl.ds(w * ROWS_PER_W, _HALF)], src_v)
    pltpu.sync_copy(ei_hbm.at[1, pl.ds(w * ROWS_PER_W, _HALF)], dst_v)
    plsc.subcore_barrier()

    _agg_ring(tbl_hbm, src_v, dst_v, bufs, gsems, ssems, acc, _HALF)
    pltpu.sync_copy(ei_hbm.at[0, pl.ds(w * ROWS_PER_W + _HALF, _HALF)], src_v)
    pltpu.sync_copy(ei_hbm.at[1, pl.ds(w * ROWS_PER_W + _HALF, _HALF)], dst_v)
    _agg_ring(tbl_hbm, src_v, dst_v, bufs, gsems, ssems, acc, _HALF)

    plsc.subcore_barrier()
    pltpu.sync_copy(acc.at[sl], out_hbm.at[c, sl])


@functools.partial(
    pl.kernel,
    out_type=jax.ShapeDtypeStruct((NCORE, NPAD, 16), jnp.float32),
    mesh=_MESH,
    scratch_types=[
        pltpu.VMEM((ROWS_PER_W, 128), jnp.int32),
        pltpu.VMEM((ROWS_PER_W, 128), jnp.int32),
        pltpu.VMEM_SHARED((NPAD, 16), jnp.float32),
        pltpu.VMEM_SHARED((NPAD, 16), jnp.float32),
    ] + [pltpu.VMEM((128, 16), jnp.float32)] * 4
      + [pltpu.SemaphoreType.DMA] * 8,
)
def _agg16(tbl_hbm, ei_hbm, out_hbm,
           src_v, dst_v, tbl_sh, acc, *rest):
    bufs = rest[:4]
    gsems, ssems = rest[4:8], rest[8:]
    c = lax.axis_index("c")
    s = lax.axis_index("s")
    w = s * NCORE + c
    sl = pl.ds(s * ROWS_PER_SUB, ROWS_PER_SUB)
    _zero_fill(bufs[0], acc, s * ROWS_PER_SUB, ROWS_PER_SUB, 16)
    pltpu.sync_copy(tbl_hbm.at[sl], tbl_sh.at[sl])
    pltpu.sync_copy(ei_hbm.at[0, pl.ds(w * ROWS_PER_W, ROWS_PER_W)], src_v)
    pltpu.sync_copy(ei_hbm.at[1, pl.ds(w * ROWS_PER_W, ROWS_PER_W)], dst_v)
    plsc.subcore_barrier()

    _agg_ring(tbl_sh, src_v, dst_v, bufs, gsems, ssems, acc, ROWS_PER_W)

    plsc.subcore_barrier()
    pltpu.sync_copy(acc.at[sl], out_hbm.at[c, sl])


_B = 2000
_GRID = N // _B


def _tc1_body(dg, x_ref, w1, g1_ref, dv_ref):
    deg = dg[0, :, 0:1] + dg[1, :, 0:1] + 1.0
    dinv = lax.rsqrt(deg)
    h = jnp.dot(x_ref[...].astype(jnp.bfloat16), w1[...].astype(jnp.bfloat16),
                preferred_element_type=jnp.float32)
    g1_ref[...] = h * dinv
    dv_ref[...] = jnp.broadcast_to(dinv, (_B, 16))


def _tc1(degp, x, W1):
    return pl.pallas_call(
        _tc1_body,
        grid=(_GRID,),
        in_specs=[
            pl.BlockSpec((2, _B, 16), lambda i: (0, i, 0)),
            pl.BlockSpec((_B, D), lambda i: (i, 0)),
            pl.BlockSpec((D, D), lambda i: (0, 0)),
        ],
        out_specs=[
            pl.BlockSpec((_B, D), lambda i: (i, 0)),
            pl.BlockSpec((_B, 16), lambda i: (i, 0)),
        ],
        out_shape=[
            jax.ShapeDtypeStruct((N, D), jnp.float32),
            jax.ShapeDtypeStruct((N, 16), jnp.float32),
        ],
    )(degp, x, W1)


def _tc2_body(ac, g1, dv, b1, w2, g2_ref):
    dinv = dv[:, 0:1]
    h = dinv * (ac[0] + ac[1] + g1[...]) + b1[...]
    h = jnp.maximum(h, 0.0)
    g2_ref[...] = dinv * jnp.dot(h.astype(jnp.bfloat16),
                                 w2[...].astype(jnp.bfloat16),
                                 preferred_element_type=jnp.float32)


def _tc2(acc1, g1, dv, b1, w2p):
    return pl.pallas_call(
        _tc2_body,
        grid=(_GRID,),
        in_specs=[
            pl.BlockSpec((2, _B, D), lambda i: (0, i, 0)),
            pl.BlockSpec((_B, D), lambda i: (i, 0)),
            pl.BlockSpec((_B, 16), lambda i: (i, 0)),
            pl.BlockSpec((1, D), lambda i: (0, 0)),
            pl.BlockSpec((D, 16), lambda i: (0, 0)),
        ],
        out_specs=pl.BlockSpec((_B, 16), lambda i: (i, 0)),
        out_shape=jax.ShapeDtypeStruct((NPAD, 16), jnp.float32),
    )(acc1, g1, dv, b1, w2p)


def _tc3_body(ac, g2, dv, b2, o_ref):
    o_ref[...] = dv[:, 0:1] * (ac[0] + ac[1] + g2[...]) + b2[...]


def _tc3(acc2, g2, dv, b2p):
    return pl.pallas_call(
        _tc3_body,
        grid=(_GRID,),
        in_specs=[
            pl.BlockSpec((2, _B, 16), lambda i: (0, i, 0)),
            pl.BlockSpec((_B, 16), lambda i: (i, 0)),
            pl.BlockSpec((_B, 16), lambda i: (i, 0)),
            pl.BlockSpec((1, 16), lambda i: (0, 0)),
        ],
        out_specs=pl.BlockSpec((_B, 16), lambda i: (i, 0)),
        out_shape=jax.ShapeDtypeStruct((N, 16), jnp.float32),
    )(acc2, g2, dv, b2p)


def kernel(x, edge_index, W1, b1, W2, b2):
    f32 = jnp.float32
    p = EPAD - E
    pad_ids = jnp.arange(p, dtype=jnp.int32)
    padblk = jnp.stack([(pad_ids * 997) % N, N + pad_ids % (NPAD - N)])
    ei = jnp.concatenate([edge_index, padblk], axis=1).reshape(2, EROWS, 128)

    w2p = jnp.zeros((D, 16), f32).at[:, :2].set(W2)
    b2p = jnp.zeros((1, 16), f32).at[0, :2].set(b2)

    degp = _deg_kernel(ei)
    g1, dv = _tc1(degp, x, W1)
    acc1 = _agg128(g1, ei)
    g2 = _tc2(acc1, g1, dv, b1.reshape(1, D), w2p)
    acc2 = _agg16(g2, ei)
    o16 = _tc3(acc2, g2, dv, b2p)
    return o16[:, :2]
